# Initial kernel scaffold; baseline (speedup 1.0000x reference)
import jax
import jax.numpy as jnp
from jax.experimental import pallas as pl


def kernel(prot_x, prot_edge_index, prot_edge_attr, lig_x, lig_edge_index, lig_edge_attr, eps, params):
    raise NotImplementedError("write your pallas kernel here")


# speedup vs baseline: 10.5100x; 10.5100x over previous
import functools

import jax
import jax.numpy as jnp
from jax import lax
from jax.experimental import pallas as pl
from jax.experimental.pallas import tpu as pltpu
from jax.experimental.pallas import tpu_sc as plsc

N_NODES = 10000
N_PAD = 10240
DUMMY = 10048
HIDDEN = 128
CHUNK = 128
NC = 2
NS = 16
ROWS_PER_TILE = N_PAD // NS


@functools.lru_cache(maxsize=None)
def _make_edge_kernel(total_chunks: int):
    cpt = total_chunks // (NC * NS)
    mesh = plsc.VectorSubcoreMesh(core_axis_name="c", subcore_axis_name="s")

    @functools.partial(
        pl.kernel,
        out_type=(
            jax.ShapeDtypeStruct((NC, N_PAD, HIDDEN), jnp.float32),
            jax.ShapeDtypeStruct((NC * N_PAD,), jnp.float32),
        ),
        mesh=mesh,
        scratch_types=[
            pltpu.VMEM((cpt, CHUNK), jnp.int32),
            pltpu.VMEM((cpt, CHUNK), jnp.int32),
            pltpu.VMEM((cpt, CHUNK), jnp.float32),
            pltpu.VMEM((CHUNK,), jnp.float32),
            pltpu.VMEM((CHUNK,), jnp.float32),
            pltpu.VMEM((CHUNK, HIDDEN), jnp.float32),
            pltpu.VMEM((ROWS_PER_TILE,), jnp.float32),
            pltpu.VMEM_SHARED((N_PAD, HIDDEN), jnp.float32),
            pltpu.VMEM_SHARED((N_PAD,), jnp.float32),
            pltpu.SemaphoreType.DMA,
            pltpu.SemaphoreType.DMA,
        ],
    )
    def edge_kernel(src_hbm, dst_hbm, se_hbm, ssrc_hbm, sdst_hbm, table_hbm,
                    acc_out, den_out,
                    src_v, dst_v, se_v, ssg_v, sdg_v, rows_v, zrow_v,
                    acc_sh, den_sh, sem_a, sem_b):
        c = lax.axis_index("c")
        s = lax.axis_index("s")
        tile = c * NS + s
        row0 = s * ROWS_PER_TILE

        zv = jnp.zeros((16,), jnp.float32)

        def zero_rows(r, _):
            for g in range(HIDDEN // 16):
                rows_v[r, pl.ds(16 * g, 16)] = zv
            return ()
        lax.fori_loop(0, CHUNK, zero_rows, ())

        def zero_strip(k, _):
            zrow_v[pl.ds(16 * k, 16)] = zv
            return ()
        lax.fori_loop(0, ROWS_PER_TILE // 16, zero_strip, ())

        for k in range(ROWS_PER_TILE // CHUNK):
            pltpu.sync_copy(rows_v, acc_sh.at[pl.ds(row0 + CHUNK * k, CHUNK)])
        pltpu.sync_copy(zrow_v, den_sh.at[pl.ds(row0, ROWS_PER_TILE)])
        plsc.subcore_barrier()

        pltpu.sync_copy(src_hbm.at[tile], src_v)
        pltpu.sync_copy(dst_hbm.at[tile], dst_v)
        pltpu.sync_copy(se_hbm.at[tile], se_v)

        def body_a(j, _):
            d1 = pltpu.async_copy(ssrc_hbm.at[src_v.at[j]], ssg_v, sem_a)
            d2 = pltpu.async_copy(sdst_hbm.at[dst_v.at[j]], sdg_v, sem_b)
            d1.wait()
            d2.wait()
            for g in range(CHUNK // 16):
                sl = pl.ds(16 * g, 16)
                a = ssg_v[sl] + sdg_v[sl] + se_v[j, sl]
                a = jnp.where(a >= 0.0, a, 0.2 * a)
                se_v[j, sl] = jnp.exp(a)
            pltpu.sync_copy(se_v.at[j], den_sh.at[dst_v.at[j]], add=True)
            return ()
        lax.fori_loop(0, cpt, body_a, ())

        def body_b(j, _):
            pltpu.async_copy(table_hbm.at[src_v.at[j]], rows_v, sem_a).wait()

            def scale_row(e, _):
                ex_g = se_v[j, pl.ds((e // 16) * 16, 16)]
                ex = lax.gather(
                    ex_g, jnp.full((16, 1), e % 16, jnp.int32),
                    lax.GatherDimensionNumbers(offset_dims=(),
                                               collapsed_slice_dims=(0,),
                                               start_index_map=(0,)),
                    slice_sizes=(1,),
                    mode=lax.GatherScatterMode.PROMISE_IN_BOUNDS)
                for g in range(HIDDEN // 16):
                    sl = pl.ds(16 * g, 16)
                    rows_v[e, sl] = rows_v[e, sl] * ex
                return ()
            lax.fori_loop(0, CHUNK, scale_row, ())
            pltpu.sync_copy(rows_v, acc_sh.at[dst_v.at[j]], add=True)
            return ()
        lax.fori_loop(0, cpt, body_b, ())

        plsc.subcore_barrier()
        pltpu.sync_copy(acc_sh.at[pl.ds(row0, ROWS_PER_TILE)],
                        acc_out.at[c, pl.ds(row0, ROWS_PER_TILE)])
        pltpu.sync_copy(den_sh.at[pl.ds(row0, ROWS_PER_TILE)],
                        den_out.at[pl.ds(c * N_PAD + row0, ROWS_PER_TILE)])

    return edge_kernel


def _proj_body(x_ref, w_ref, a_ref, h_ref, s_ref):
    h = jnp.dot(x_ref[...], w_ref[...], preferred_element_type=jnp.float32)
    h_ref[...] = h
    s_ref[...] = jnp.dot(h, a_ref[...], preferred_element_type=jnp.float32)


def _proj(x_pad, w, a2):
    return pl.pallas_call(
        _proj_body,
        out_shape=(
            jax.ShapeDtypeStruct((N_PAD, HIDDEN), jnp.float32),
            jax.ShapeDtypeStruct((N_PAD, 2), jnp.float32),
        ),
    )(x_pad, w, a2)


def _se_body(ea_ref, we_ref, ae_ref, out_ref):
    v = jnp.dot(we_ref[...], ae_ref[...], preferred_element_type=jnp.float32)
    vv = jnp.tile(v, (8, 8))
    ri = lax.broadcasted_iota(jnp.int32, (128, 16), 0) // 16
    ci = lax.broadcasted_iota(jnp.int32, (128, 16), 1) // 2
    wblk = jnp.where(ri == ci, vv, 0.0)
    out_ref[...] = jnp.dot(ea_ref[...], wblk, preferred_element_type=jnp.float32)


def _se_both(edge_attr_pad, we2, ae2):
    e_pad = edge_attr_pad.shape[0]
    packed = edge_attr_pad.reshape(e_pad // 8, 128)
    out = pl.pallas_call(
        _se_body,
        out_shape=jax.ShapeDtypeStruct((e_pad // 8, 16), jnp.float32),
    )(packed, we2, ae2)
    return out.reshape(e_pad, 2)


def _combine_mid_body(acc_ref, den_ref, b_ref, g_ref, beta_ref, w_ref, a_ref,
                      h_ref, s_ref):
    acc = acc_ref[0] + acc_ref[1]
    den = den_ref[0, :, :] + den_ref[1, :, :]
    y = acc[:N_NODES] / (den[:N_NODES] + 1e-16) + b_ref[...]
    m = jnp.mean(y, axis=0, keepdims=True)
    v = jnp.mean((y - m) ** 2, axis=0, keepdims=True)
    y = (y - m) / jnp.sqrt(v + 1e-5) * g_ref[...] + beta_ref[...]
    y = jnp.maximum(y, 0.0)
    h = jnp.dot(y, w_ref[...], preferred_element_type=jnp.float32)
    h_ref[:N_NODES] = h
    h_ref[N_NODES:] = jnp.zeros((N_PAD - N_NODES, HIDDEN), jnp.float32)
    s_ref[:N_NODES] = jnp.dot(h, a_ref[...], preferred_element_type=jnp.float32)
    s_ref[N_NODES:] = jnp.zeros((N_PAD - N_NODES, 2), jnp.float32)


def _combine_mid(acc, den3, b, gamma, beta, w, a2):
    return pl.pallas_call(
        _combine_mid_body,
        out_shape=(
            jax.ShapeDtypeStruct((N_PAD, HIDDEN), jnp.float32),
            jax.ShapeDtypeStruct((N_PAD, 2), jnp.float32),
        ),
    )(acc, den3, b, gamma, beta, w, a2)


def _combine_final_body(acc_ref, den_ref, b_ref, g_ref, beta_ref, out_ref):
    acc = acc_ref[0] + acc_ref[1]
    den = den_ref[0, :, :] + den_ref[1, :, :]
    y = acc[:N_NODES] / (den[:N_NODES] + 1e-16) + b_ref[...]
    m = jnp.mean(y, axis=0, keepdims=True)
    v = jnp.mean((y - m) ** 2, axis=0, keepdims=True)
    y = (y - m) / jnp.sqrt(v + 1e-5) * g_ref[...] + beta_ref[...]
    y = jnp.maximum(y, 0.0)
    out_ref[...] = jnp.mean(y, axis=0, keepdims=True)


def _combine_final(acc, den3, b, gamma, beta):
    return pl.pallas_call(
        _combine_final_body,
        out_shape=jax.ShapeDtypeStruct((1, HIDDEN), jnp.float32),
    )(acc, den3, b, gamma, beta)


def _head_body(gp_ref, gl_ref, wmu_ref, bmu_ref, wlv_ref, blv_ref,
               wdec_ref, bdec_ref, eps_ref, nf_ref, mu_ref, lv_ref):
    hcat = jnp.concatenate([gp_ref[...], gl_ref[...]], axis=1)
    mu = jnp.dot(hcat, wmu_ref[...], preferred_element_type=jnp.float32) + bmu_ref[...]
    lv = jnp.dot(hcat, wlv_ref[...], preferred_element_type=jnp.float32) + blv_ref[...]
    z = mu + jnp.exp(0.5 * lv) * eps_ref[...]
    cond = jnp.concatenate([z, gp_ref[...]], axis=1)
    nf = jnp.dot(cond, wdec_ref[...], preferred_element_type=jnp.float32) + bdec_ref[...]
    nf_ref[...] = jnp.broadcast_to(nf[:, None, :], nf_ref.shape)
    mu_ref[...] = mu
    lv_ref[...] = lv


def _head(gp, gl, p, eps):
    return pl.pallas_call(
        _head_body,
        out_shape=(
            jax.ShapeDtypeStruct((1, 48, HIDDEN), jnp.float32),
            jax.ShapeDtypeStruct((1, 64), jnp.float32),
            jax.ShapeDtypeStruct((1, 64), jnp.float32),
        ),
    )(gp, gl, p["W_mu"], p["b_mu"][None, :], p["W_lv"], p["b_lv"][None, :],
      p["W_dec"], p["b_dec"][None, :], eps[None, :])


def _pad_edges(edge_index, se2, total_chunks):
    e_pad = total_chunks * CHUNK
    e = edge_index.shape[1]
    cpt = total_chunks // (NC * NS)
    src = jnp.full((e_pad,), DUMMY, jnp.int32).at[:e].set(edge_index[0].astype(jnp.int32))
    dst = jnp.full((e_pad,), DUMMY, jnp.int32).at[:e].set(edge_index[1].astype(jnp.int32))
    src = src.reshape(NC * NS, cpt, CHUNK)
    dst = dst.reshape(NC * NS, cpt, CHUNK)
    se2 = se2.reshape(NC * NS, cpt, CHUNK, 2)
    return src, dst, se2


def _encoder(x, edge_index, edge_attr, p1, p2, bn1_g, bn1_b, bn2_g, bn2_b,
             total_chunks):
    e_pad = total_chunks * CHUNK
    edge_kernel = _make_edge_kernel(total_chunks)

    x_pad = jnp.zeros((N_PAD, x.shape[1]), jnp.float32).at[:N_NODES].set(x)
    ea_pad = jnp.zeros((e_pad, edge_attr.shape[1]), jnp.float32).at[:edge_attr.shape[0]].set(edge_attr)

    a2_1 = jnp.stack([p1["a_src"], p1["a_dst"]], axis=1)
    a2_2 = jnp.stack([p2["a_src"], p2["a_dst"]], axis=1)
    we2 = jnp.concatenate([p1["We"], p2["We"]], axis=1)
    ae2 = jnp.zeros((2 * HIDDEN, 2), jnp.float32)
    ae2 = ae2.at[:HIDDEN, 0].set(p1["a_e"]).at[HIDDEN:, 1].set(p2["a_e"])
    se_both = _se_both(ea_pad, we2, ae2)

    src, dst, se3 = _pad_edges(edge_index, se_both, total_chunks)

    h1, s1 = _proj(x_pad, p1["W"], a2_1)
    ssrc1 = s1[:, 0]
    sdst1 = s1[:, 1]
    se1 = se3[:, :, :, 0]
    acc1, den1 = edge_kernel(src, dst, se1, ssrc1, sdst1, h1)
    den1_3 = den1.reshape(NC, N_PAD, 1)

    h2, s2 = _combine_mid(acc1, den1_3, p1["b"][None, :], bn1_g[None, :],
                          bn1_b[None, :], p2["W"], a2_2)
    ssrc2 = s2[:, 0]
    sdst2 = s2[:, 1]
    se2 = se3[:, :, :, 1]
    acc2, den2 = edge_kernel(src, dst, se2, ssrc2, sdst2, h2)
    den2_3 = den2.reshape(NC, N_PAD, 1)

    g = _combine_final(acc2, den2_3, p2["b"][None, :], bn2_g[None, :],
                       bn2_b[None, :])
    return g


def kernel(prot_x, prot_edge_index, prot_edge_attr, lig_x, lig_edge_index,
           lig_edge_attr, eps, params):
    p = params
    gp = _encoder(prot_x, prot_edge_index, prot_edge_attr,
                  p["p_conv1"], p["p_conv2"], p["p_bn1_g"], p["p_bn1_b"],
                  p["p_bn2_g"], p["p_bn2_b"], total_chunks=2528)
    gl = _encoder(lig_x, lig_edge_index, lig_edge_attr,
                  p["l_conv1"], p["l_conv2"], p["l_bn1_g"], p["l_bn1_b"],
                  p["l_bn2_g"], p["l_bn2_b"], total_chunks=1280)
    nf, mu, lv = _head(gp, gl, p, eps)
    return (nf, mu[0], lv[0])


# speedup vs baseline: 12.5937x; 1.1983x over previous
import functools

import jax
import jax.numpy as jnp
from jax import lax
from jax.experimental import pallas as pl
from jax.experimental.pallas import tpu as pltpu
from jax.experimental.pallas import tpu_sc as plsc

N_NODES = 10000
N_PAD = 10240
DUMMY = 10048
HIDDEN = 128
CHUNK = 128
NC = 2
NS = 16
ROWS_PER_TILE = N_PAD // NS


def _splat(vec16, lane):
    return lax.gather(
        vec16, jnp.full((16, 1), lane, jnp.int32),
        lax.GatherDimensionNumbers(offset_dims=(), collapsed_slice_dims=(0,),
                                   start_index_map=(0,)),
        slice_sizes=(1,),
        mode=lax.GatherScatterMode.PROMISE_IN_BOUNDS)


@functools.lru_cache(maxsize=None)
def _make_edge_kernel(total_chunks: int):
    cpt = total_chunks // (NC * NS)
    mesh = plsc.VectorSubcoreMesh(core_axis_name="c", subcore_axis_name="s")

    @functools.partial(
        pl.kernel,
        out_type=(
            jax.ShapeDtypeStruct((NC, N_PAD, HIDDEN), jnp.float32),
            jax.ShapeDtypeStruct((NC * N_PAD,), jnp.float32),
        ),
        mesh=mesh,
        scratch_types=[
            pltpu.VMEM((4, CHUNK), jnp.int32),
            pltpu.VMEM((4, CHUNK), jnp.int32),
            pltpu.VMEM((4, CHUNK), jnp.float32),
            pltpu.VMEM((2, CHUNK), jnp.float32),
            pltpu.VMEM((2, CHUNK), jnp.float32),
            pltpu.VMEM((2, CHUNK), jnp.float32),
            pltpu.VMEM((2, CHUNK, HIDDEN), jnp.float32),
            pltpu.VMEM((ROWS_PER_TILE,), jnp.float32),
            pltpu.VMEM_SHARED((N_PAD, HIDDEN), jnp.float32),
            pltpu.VMEM_SHARED((N_PAD,), jnp.float32),
            pltpu.SemaphoreType.DMA, pltpu.SemaphoreType.DMA,
            pltpu.SemaphoreType.DMA, pltpu.SemaphoreType.DMA,
            pltpu.SemaphoreType.DMA, pltpu.SemaphoreType.DMA,
            pltpu.SemaphoreType.DMA, pltpu.SemaphoreType.DMA,
        ],
    )
    def edge_kernel(src_hbm, dst_hbm, se_hbm, ssrc_hbm, sdst_hbm, table_hbm,
                    acc_out, den_out,
                    src4, dst4, se4, ssg2, sdg2, ex2, rows2, zrow_v,
                    acc_sh, den_sh,
                    ls0, ls1, ls2, ls3, gs0, gs1, ss0, ss1):
        lsems = [ls0, ls1, ls2, ls3]
        gsems = [gs0, gs1]
        ssems = [ss0, ss1]
        c = lax.axis_index("c")
        s = lax.axis_index("s")
        tile = c * NS + s
        row0 = s * ROWS_PER_TILE
        zv = jnp.zeros((16,), jnp.float32)

        rows0 = rows2.at[0]

        def zero_rows(r, _):
            for g in range(HIDDEN // 16):
                rows0[r, pl.ds(16 * g, 16)] = zv
            return ()
        lax.fori_loop(0, CHUNK, zero_rows, ())

        def zero_strip(k, _):
            zrow_v[pl.ds(16 * k, 16)] = zv
            return ()
        lax.fori_loop(0, ROWS_PER_TILE // 16, zero_strip, ())

        for k in range(ROWS_PER_TILE // CHUNK):
            pltpu.sync_copy(rows0, acc_sh.at[pl.ds(row0 + CHUNK * k, CHUNK)])
        pltpu.sync_copy(zrow_v, den_sh.at[pl.ds(row0, ROWS_PER_TILE)])
        plsc.subcore_barrier()

        def fire_lin(k, slot):
            base = (tile * cpt + k) * CHUNK
            sem = lsems[slot]
            pltpu.async_copy(src_hbm.at[pl.ds(base, CHUNK)], src4.at[slot], sem)
            pltpu.async_copy(dst_hbm.at[pl.ds(base, CHUNK)], dst4.at[slot], sem)
            pltpu.async_copy(se_hbm.at[pl.ds(base, CHUNK)], se4.at[slot], sem)

        def drain_lin(slot):
            sem = lsems[slot]
            pltpu.make_async_copy(src_hbm.at[pl.ds(0, CHUNK)], src4.at[slot], sem).wait()
            pltpu.make_async_copy(src_hbm.at[pl.ds(0, CHUNK)], dst4.at[slot], sem).wait()
            pltpu.make_async_copy(se_hbm.at[pl.ds(0, CHUNK)], se4.at[slot], sem).wait()

        def fire_gather(idx_slot, b):
            sem = gsems[b]
            pltpu.async_copy(table_hbm.at[src4.at[idx_slot]], rows2.at[b], sem)
            pltpu.async_copy(ssrc_hbm.at[src4.at[idx_slot]], ssg2.at[b], sem)
            pltpu.async_copy(sdst_hbm.at[dst4.at[idx_slot]], sdg2.at[b], sem)

        def drain_gather(b):
            sem = gsems[b]
            pltpu.make_async_copy(table_hbm.at[pl.ds(0, CHUNK)], rows2.at[b], sem).wait()
            pltpu.make_async_copy(ssrc_hbm.at[pl.ds(0, CHUNK)], ssg2.at[b], sem).wait()
            pltpu.make_async_copy(ssrc_hbm.at[pl.ds(0, CHUNK)], sdg2.at[b], sem).wait()

        def fire_scatter(b, idx_slot):
            sem = ssems[b]
            pltpu.async_copy(rows2.at[b], acc_sh.at[dst4.at[idx_slot]], sem, add=True)
            pltpu.async_copy(ex2.at[b], den_sh.at[dst4.at[idx_slot]], sem, add=True)

        def drain_scatter(b):
            sem = ssems[b]
            pltpu.make_async_copy(table_hbm.at[pl.ds(0, CHUNK)], rows2.at[b], sem).wait()
            pltpu.make_async_copy(ssrc_hbm.at[pl.ds(0, CHUNK)], ex2.at[b], sem).wait()

        def sub(j, q, sc_drain=True, nxt_gather=True, nxt_lin=True):
            b = q % 2
            if sc_drain:
                drain_scatter(1 - b)
            if nxt_gather:
                drain_lin((q + 1) % 4)
                fire_gather((q + 1) % 4, 1 - b)
            if nxt_lin:
                fire_lin(j + 2, (q + 2) % 4)
            drain_gather(b)
            for g in range(CHUNK // 16):
                sl = pl.ds(16 * g, 16)
                a = ssg2[b, sl] + sdg2[b, sl] + se4[q, sl]
                a = jnp.where(a >= 0.0, a, 0.2 * a)
                ex2[b, sl] = jnp.exp(a)
            rows_b = rows2.at[b]

            def scale4(i, _):
                ex_g = ex2[b, pl.ds((i // 4) * 16, 16)]
                for u in range(4):
                    exs = _splat(ex_g, (i % 4) * 4 + u)
                    e = 4 * i + u
                    for g in range(HIDDEN // 16):
                        sl = pl.ds(16 * g, 16)
                        rows_b[e, sl] = rows_b[e, sl] * exs
                return ()
            lax.fori_loop(0, CHUNK // 4, scale4, ())
            fire_scatter(b, q)

        fire_lin(0, 0)
        fire_lin(1, 1)
        drain_lin(0)
        fire_gather(0, 0)
        sub(0, 0, sc_drain=False)
        sub(1, 1)
        sub(2, 2)
        sub(3, 3)

        def grp(j4, _):
            for q in range(4):
                sub(4 * j4 + q, q)
            return ()
        lax.fori_loop(1, cpt // 4 - 1, grp, ())

        jb = cpt - 4
        sub(jb + 0, 0)
        sub(jb + 1, 1)
        sub(jb + 2, 2, nxt_lin=False)
        sub(jb + 3, 3, nxt_lin=False, nxt_gather=False)
        drain_scatter((cpt - 1) % 2)

        plsc.subcore_barrier()
        pltpu.sync_copy(acc_sh.at[pl.ds(row0, ROWS_PER_TILE)],
                        acc_out.at[c, pl.ds(row0, ROWS_PER_TILE)])
        pltpu.sync_copy(den_sh.at[pl.ds(row0, ROWS_PER_TILE)],
                        den_out.at[pl.ds(c * N_PAD + row0, ROWS_PER_TILE)])

    return edge_kernel


def _proj_body(x_ref, w_ref, a_ref, h_ref, s_ref):
    h = jnp.dot(x_ref[...], w_ref[...], preferred_element_type=jnp.float32)
    h_ref[...] = h
    s_ref[...] = jnp.dot(h, a_ref[...], preferred_element_type=jnp.float32)


def _proj(x_pad, w, a2):
    return pl.pallas_call(
        _proj_body,
        out_shape=(
            jax.ShapeDtypeStruct((N_PAD, HIDDEN), jnp.float32),
            jax.ShapeDtypeStruct((N_PAD, 2), jnp.float32),
        ),
    )(x_pad, w, a2)


def _se_body(ea_ref, we_ref, ae_ref, out_ref):
    v = jnp.dot(we_ref[...], ae_ref[...], preferred_element_type=jnp.float32)
    vv = jnp.tile(v, (8, 8))
    ri = lax.broadcasted_iota(jnp.int32, (128, 16), 0) // 16
    ci = lax.broadcasted_iota(jnp.int32, (128, 16), 1) // 2
    wblk = jnp.where(ri == ci, vv, 0.0)
    out_ref[...] = jnp.dot(ea_ref[...], wblk, preferred_element_type=jnp.float32)


def _se_both(edge_attr_pad, we2, ae2):
    e_pad = edge_attr_pad.shape[0]
    packed = edge_attr_pad.reshape(e_pad // 8, 128)
    out = pl.pallas_call(
        _se_body,
        out_shape=jax.ShapeDtypeStruct((e_pad // 8, 16), jnp.float32),
    )(packed, we2, ae2)
    return out.reshape(e_pad, 2)


def _combine_mid_body(acc_ref, den_ref, b_ref, g_ref, beta_ref, w_ref, a_ref,
                      h_ref, s_ref):
    acc = acc_ref[0] + acc_ref[1]
    den = den_ref[0, :, :] + den_ref[1, :, :]
    y = acc[:N_NODES] / (den[:N_NODES] + 1e-16) + b_ref[...]
    m = jnp.mean(y, axis=0, keepdims=True)
    v = jnp.mean((y - m) ** 2, axis=0, keepdims=True)
    y = (y - m) / jnp.sqrt(v + 1e-5) * g_ref[...] + beta_ref[...]
    y = jnp.maximum(y, 0.0)
    h = jnp.dot(y, w_ref[...], preferred_element_type=jnp.float32)
    h_ref[:N_NODES] = h
    h_ref[N_NODES:] = jnp.zeros((N_PAD - N_NODES, HIDDEN), jnp.float32)
    s_ref[:N_NODES] = jnp.dot(h, a_ref[...], preferred_element_type=jnp.float32)
    s_ref[N_NODES:] = jnp.zeros((N_PAD - N_NODES, 2), jnp.float32)


def _combine_mid(acc, den3, b, gamma, beta, w, a2):
    return pl.pallas_call(
        _combine_mid_body,
        out_shape=(
            jax.ShapeDtypeStruct((N_PAD, HIDDEN), jnp.float32),
            jax.ShapeDtypeStruct((N_PAD, 2), jnp.float32),
        ),
    )(acc, den3, b, gamma, beta, w, a2)


def _combine_final_body(acc_ref, den_ref, b_ref, g_ref, beta_ref, out_ref):
    acc = acc_ref[0] + acc_ref[1]
    den = den_ref[0, :, :] + den_ref[1, :, :]
    y = acc[:N_NODES] / (den[:N_NODES] + 1e-16) + b_ref[...]
    m = jnp.mean(y, axis=0, keepdims=True)
    v = jnp.mean((y - m) ** 2, axis=0, keepdims=True)
    y = (y - m) / jnp.sqrt(v + 1e-5) * g_ref[...] + beta_ref[...]
    y = jnp.maximum(y, 0.0)
    out_ref[...] = jnp.mean(y, axis=0, keepdims=True)


def _combine_final(acc, den3, b, gamma, beta):
    return pl.pallas_call(
        _combine_final_body,
        out_shape=jax.ShapeDtypeStruct((1, HIDDEN), jnp.float32),
    )(acc, den3, b, gamma, beta)


def _head_body(gp_ref, gl_ref, wmu_ref, bmu_ref, wlv_ref, blv_ref,
               wdec_ref, bdec_ref, eps_ref, nf_ref, mu_ref, lv_ref):
    hcat = jnp.concatenate([gp_ref[...], gl_ref[...]], axis=1)
    mu = jnp.dot(hcat, wmu_ref[...], preferred_element_type=jnp.float32) + bmu_ref[...]
    lv = jnp.dot(hcat, wlv_ref[...], preferred_element_type=jnp.float32) + blv_ref[...]
    z = mu + jnp.exp(0.5 * lv) * eps_ref[...]
    cond = jnp.concatenate([z, gp_ref[...]], axis=1)
    nf = jnp.dot(cond, wdec_ref[...], preferred_element_type=jnp.float32) + bdec_ref[...]
    nf_ref[...] = jnp.broadcast_to(nf[:, None, :], nf_ref.shape)
    mu_ref[...] = mu
    lv_ref[...] = lv


def _head(gp, gl, p, eps):
    return pl.pallas_call(
        _head_body,
        out_shape=(
            jax.ShapeDtypeStruct((1, 48, HIDDEN), jnp.float32),
            jax.ShapeDtypeStruct((1, 64), jnp.float32),
            jax.ShapeDtypeStruct((1, 64), jnp.float32),
        ),
    )(gp, gl, p["W_mu"], p["b_mu"][None, :], p["W_lv"], p["b_lv"][None, :],
      p["W_dec"], p["b_dec"][None, :], eps[None, :])


def _pad_edges(edge_index, total_chunks):
    e_pad = total_chunks * CHUNK
    e = edge_index.shape[1]
    src = jnp.full((e_pad,), DUMMY, jnp.int32).at[:e].set(edge_index[0].astype(jnp.int32))
    dst = jnp.full((e_pad,), DUMMY, jnp.int32).at[:e].set(edge_index[1].astype(jnp.int32))
    return src, dst


def _encoder(x, edge_index, edge_attr, p1, p2, bn1_g, bn1_b, bn2_g, bn2_b,
             total_chunks):
    e_pad = total_chunks * CHUNK
    edge_kernel = _make_edge_kernel(total_chunks)

    x_pad = jnp.zeros((N_PAD, x.shape[1]), jnp.float32).at[:N_NODES].set(x)
    ea_pad = jnp.zeros((e_pad, edge_attr.shape[1]), jnp.float32).at[:edge_attr.shape[0]].set(edge_attr)

    a2_1 = jnp.stack([p1["a_src"], p1["a_dst"]], axis=1)
    a2_2 = jnp.stack([p2["a_src"], p2["a_dst"]], axis=1)
    we2 = jnp.concatenate([p1["We"], p2["We"]], axis=1)
    ae2 = jnp.zeros((2 * HIDDEN, 2), jnp.float32)
    ae2 = ae2.at[:HIDDEN, 0].set(p1["a_e"]).at[HIDDEN:, 1].set(p2["a_e"])
    se_both = _se_both(ea_pad, we2, ae2)

    src, dst = _pad_edges(edge_index, total_chunks)

    h1, s1 = _proj(x_pad, p1["W"], a2_1)
    ssrc1 = s1[:, 0]
    sdst1 = s1[:, 1]
    se1 = se_both[:, 0]
    acc1, den1 = edge_kernel(src, dst, se1, ssrc1, sdst1, h1)
    den1_3 = den1.reshape(NC, N_PAD, 1)

    h2, s2 = _combine_mid(acc1, den1_3, p1["b"][None, :], bn1_g[None, :],
                          bn1_b[None, :], p2["W"], a2_2)
    ssrc2 = s2[:, 0]
    sdst2 = s2[:, 1]
    se2 = se_both[:, 1]
    acc2, den2 = edge_kernel(src, dst, se2, ssrc2, sdst2, h2)
    den2_3 = den2.reshape(NC, N_PAD, 1)

    g = _combine_final(acc2, den2_3, p2["b"][None, :], bn2_g[None, :],
                       bn2_b[None, :])
    return g


def kernel(prot_x, prot_edge_index, prot_edge_attr, lig_x, lig_edge_index,
           lig_edge_attr, eps, params):
    p = params
    gp = _encoder(prot_x, prot_edge_index, prot_edge_attr,
                  p["p_conv1"], p["p_conv2"], p["p_bn1_g"], p["p_bn1_b"],
                  p["p_bn2_g"], p["p_bn2_b"], total_chunks=2560)
    gl = _encoder(lig_x, lig_edge_index, lig_edge_attr,
                  p["l_conv1"], p["l_conv2"], p["l_bn1_g"], p["l_bn1_b"],
                  p["l_bn2_g"], p["l_bn2_b"], total_chunks=1280)
    nf, mu, lv = _head(gp, gl, p, eps)
    return (nf, mu[0], lv[0])
